# Initial kernel scaffold; baseline (speedup 1.0000x reference)
import jax
import jax.numpy as jnp
from jax.experimental import pallas as pl


def kernel(feat_vul, feat_weakness_name, feat_other, src_w2v, dst_w2v, src_o2v, dst_o2v, src_v2w, dst_v2w, src_v2o, dst_v2o, W_w2v, W_o2v, W_v2w, W_v2o, Wn_vul, bn_vul, Wn_weakness_name, bn_weakness_name, Wn_other, bn_other):
    raise NotImplementedError("write your pallas kernel here")


# speedup vs baseline: 1.4645x; 1.4645x over previous
import jax
import jax.numpy as jnp
from jax.experimental import pallas as pl


def _mm_kernel(x_ref, w_ref, o_ref):
    o_ref[...] = jax.lax.dot_general(
        x_ref[...], w_ref[...], (((1,), (1,)), ((), ())),
        preferred_element_type=jnp.float32)


def _mm(x, w, bm=1000):
    m, d = x.shape
    return pl.pallas_call(
        _mm_kernel,
        grid=(m // bm,),
        in_specs=[pl.BlockSpec((bm, d), lambda i: (i, 0)),
                  pl.BlockSpec((d, d), lambda i: (0, 0))],
        out_specs=pl.BlockSpec((bm, d), lambda i: (i, 0)),
        out_shape=jax.ShapeDtypeStruct((m, d), jnp.float32),
    )(x, w)


def _rowscale(h, c, eps=1e-8):
    n = jnp.maximum(jnp.linalg.norm(h, axis=-1, keepdims=True), eps)
    return h * jnp.sqrt(c / n)


def _vhat(t, eps=1e-8):
    n = jnp.maximum(jnp.linalg.norm(t, axis=-1, keepdims=True), eps)
    return t / n


def _msg(q, vh, src, dst, n_dst):
    qs = q[src]
    s = jnp.sum(qs * vh[dst], axis=-1)
    return jax.ops.segment_sum(s[:, None] * qs, dst, num_segments=n_dst)


def kernel(feat_vul, feat_weakness_name, feat_other, src_w2v, dst_w2v,
           src_o2v, dst_o2v, src_v2w, dst_v2w, src_v2o, dst_v2o,
           W_w2v, W_o2v, W_v2w, W_v2o,
           Wn_vul, bn_vul, Wn_weakness_name, bn_weakness_name,
           Wn_other, bn_other):
    nv, nw, no = feat_vul.shape[0], feat_weakness_name.shape[0], feat_other.shape[0]
    ht_vul = _mm(feat_vul, Wn_vul) + bn_vul
    ht_w = _mm(feat_weakness_name, Wn_weakness_name) + bn_weakness_name
    ht_o = _mm(feat_other, Wn_other) + bn_other

    q_w2v = _rowscale(_mm(feat_weakness_name, W_w2v), 0.6)
    q_o2v = _rowscale(_mm(feat_other, W_o2v), 0.4)
    q_v2w = _rowscale(_mm(feat_vul, W_v2w), 1.0)
    q_v2o = _rowscale(_mm(feat_vul, W_v2o), 1.0)

    vh_vul, vh_w, vh_o = _vhat(ht_vul), _vhat(ht_w), _vhat(ht_o)

    h_vul = (_msg(q_w2v, vh_vul, src_w2v, dst_w2v, nv)
             + _msg(q_o2v, vh_vul, src_o2v, dst_o2v, nv))
    h_w = _msg(q_v2w, vh_w, src_v2w, dst_v2w, nw)
    h_o = _msg(q_v2o, vh_o, src_v2o, dst_v2o, no)

    out_vul = jnp.concatenate([ht_vul, h_vul], axis=1)
    out_w = jnp.concatenate([ht_w, h_w], axis=1)
    out_o = jnp.concatenate([ht_o, h_o], axis=1)
    return (out_vul, out_w, out_o)
